# baseline (speedup 1.0000x reference)
import functools

import jax
import jax.numpy as jnp
import numpy as np
from jax.experimental import pallas as pl
from jax.sharding import Mesh, PartitionSpec


def _ktakes_kernel(k, g_ref, out_ref):
    g = g_ref[...]
    b = jax.lax.bitcast_convert_type(g, jnp.uint32)
    u = jnp.where(b >= jnp.uint32(0x80000000), ~b, b | jnp.uint32(0x80000000))
    rows = g.shape[0]
    thr = jnp.zeros((rows, 1), jnp.uint32)
    for bit in range(31, -1, -1):
        cand = thr | jnp.uint32(1 << bit)
        cnt = jnp.sum((u < cand).astype(jnp.int32), axis=1, keepdims=True)
        thr = jnp.where(cnt < k, cand, thr)
    out_ref[...] = jnp.where(u <= thr, jnp.float32(0.0), g)


def _ktakes(g):
    B, N = g.shape
    k = int(N * 0.5)
    return pl.pallas_call(
        functools.partial(_ktakes_kernel, k),
        out_shape=jax.ShapeDtypeStruct((B, N), g.dtype),
    )(g)


def kernel(g):
    devs = jax.devices()
    if len(devs) >= 2 and g.shape[0] % 2 == 0:
        mesh = Mesh(np.array(devs[:2]), ("x",))
        f = jax.shard_map(
            _ktakes,
            mesh=mesh,
            in_specs=PartitionSpec("x", None),
            out_specs=PartitionSpec("x", None),
            check_vma=False,
        )
        return f(g)
    return _ktakes(g)


# speedup vs baseline: 38.2826x; 38.2826x over previous
import functools

import jax
import jax.numpy as jnp
from jax.experimental import pallas as pl


def _count16(mask_i16):
    x = mask_i16
    while x.shape[1] > 512:
        half = x.shape[1] // 2
        x = x[:, :half] + x[:, half:]
    return jnp.sum(x.astype(jnp.int32), axis=1, keepdims=True)


def _ktakes_kernel(k, g_ref, out_ref):
    g = g_ref[...]
    b = jax.lax.bitcast_convert_type(g, jnp.uint32)
    u = jnp.where(b >= jnp.uint32(0x80000000), ~b, b | jnp.uint32(0x80000000))
    rows = g.shape[0]
    one = jnp.int16(1)
    zero = jnp.int16(0)

    h = ((u >> jnp.uint32(16)) ^ jnp.uint32(0x8000)).astype(jnp.int16)
    thr = jnp.zeros((rows, 1), jnp.uint32)
    for bit in range(15, -1, -1):
        cand = thr | jnp.uint32(1 << bit)
        cs = (cand ^ jnp.uint32(0x8000)).astype(jnp.int16)
        cnt = _count16(jnp.where(h < cs, one, zero))
        thr = jnp.where(cnt < k, cand, thr)

    thr_s = (thr ^ jnp.uint32(0x8000)).astype(jnp.int16)
    cnt_base = _count16(jnp.where(h < thr_s, one, zero))
    lo = jnp.where(h == thr_s,
                   ((u & jnp.uint32(0xFFFF)) ^ jnp.uint32(0x8000)).astype(jnp.int16),
                   jnp.int16(0x7FFF))
    k2 = k - cnt_base

    thr_l = jnp.zeros((rows, 1), jnp.uint32)
    for bit in range(15, -1, -1):
        cand = thr_l | jnp.uint32(1 << bit)
        cs = (cand ^ jnp.uint32(0x8000)).astype(jnp.int16)
        cnt = _count16(jnp.where(lo < cs, one, zero))
        thr_l = jnp.where(cnt < k2, cand, thr_l)

    t = (thr << jnp.uint32(16)) | thr_l
    out_ref[...] = jnp.where(u <= t, jnp.float32(0.0), g)


def kernel(g):
    B, N = g.shape
    k = int(N * 0.5)
    return pl.pallas_call(
        functools.partial(_ktakes_kernel, k),
        out_shape=jax.ShapeDtypeStruct((B, N), g.dtype),
    )(g)


# speedup vs baseline: 38.9539x; 1.0175x over previous
import functools

import jax
import jax.numpy as jnp
from jax.experimental import pallas as pl

_CHUNK = 256
_NACC = 2


def _count_lt16_2(va, ca, vb, cb):
    rows, n = va.shape
    one = jnp.int16(1)
    zero = jnp.int16(0)
    acca = [jnp.zeros((rows, _CHUNK), jnp.int16) for _ in range(_NACC)]
    accb = [jnp.zeros((rows, _CHUNK), jnp.int16) for _ in range(_NACC)]
    for i, c in enumerate(range(0, n, _CHUNK)):
        sl = slice(c, c + _CHUNK)
        acca[i % _NACC] = acca[i % _NACC] + jnp.where(va[:, sl] < ca, one, zero)
        accb[i % _NACC] = accb[i % _NACC] + jnp.where(vb[:, sl] < cb, one, zero)
    while len(acca) > 1:
        acca = [x + y for x, y in zip(acca[::2], acca[1::2])]
        accb = [x + y for x, y in zip(accb[::2], accb[1::2])]
    cnta = jnp.sum(acca[0].astype(jnp.int32), axis=1, keepdims=True)
    cntb = jnp.sum(accb[0].astype(jnp.int32), axis=1, keepdims=True)
    return cnta, cntb


def _ktakes_kernel(k, g_ref, out_ref):
    g = g_ref[...]
    b = jax.lax.bitcast_convert_type(g, jnp.uint32)
    u = jnp.where(b >= jnp.uint32(0x80000000), ~b, b | jnp.uint32(0x80000000))
    rows = g.shape[0]
    half = rows // 2
    xk = jnp.uint32(0x8000)

    h = ((u >> jnp.uint32(16)) ^ xk).astype(jnp.int16)
    ha, hb = h[:half], h[half:]
    thra = jnp.zeros((half, 1), jnp.uint32)
    thrb = jnp.zeros((rows - half, 1), jnp.uint32)
    for bit in range(15, -1, -1):
        canda = thra | jnp.uint32(1 << bit)
        candb = thrb | jnp.uint32(1 << bit)
        cnta, cntb = _count_lt16_2(ha, (canda ^ xk).astype(jnp.int16),
                                   hb, (candb ^ xk).astype(jnp.int16))
        thra = jnp.where(cnta < k, canda, thra)
        thrb = jnp.where(cntb < k, candb, thrb)

    thra_s = (thra ^ xk).astype(jnp.int16)
    thrb_s = (thrb ^ xk).astype(jnp.int16)
    basea, baseb = _count_lt16_2(ha, thra_s, hb, thrb_s)
    lo = ((u & jnp.uint32(0xFFFF)) ^ xk).astype(jnp.int16)
    loa = jnp.where(ha == thra_s, lo[:half], jnp.int16(0x7FFF))
    lob = jnp.where(hb == thrb_s, lo[half:], jnp.int16(0x7FFF))
    k2a = k - basea
    k2b = k - baseb

    thrla = jnp.zeros((half, 1), jnp.uint32)
    thrlb = jnp.zeros((rows - half, 1), jnp.uint32)
    for bit in range(15, -1, -1):
        canda = thrla | jnp.uint32(1 << bit)
        candb = thrlb | jnp.uint32(1 << bit)
        cnta, cntb = _count_lt16_2(loa, (canda ^ xk).astype(jnp.int16),
                                   lob, (candb ^ xk).astype(jnp.int16))
        thrla = jnp.where(cnta < k2a, canda, thrla)
        thrlb = jnp.where(cntb < k2b, candb, thrlb)

    t = jnp.concatenate([(thra << jnp.uint32(16)) | thrla,
                         (thrb << jnp.uint32(16)) | thrlb], axis=0)
    out_ref[...] = jnp.where(u <= t, jnp.float32(0.0), g)


def kernel(g):
    B, N = g.shape
    k = int(N * 0.5)
    return pl.pallas_call(
        functools.partial(_ktakes_kernel, k),
        out_shape=jax.ShapeDtypeStruct((B, N), g.dtype),
    )(g)


# speedup vs baseline: 45.5648x; 1.1697x over previous
import functools

import jax
import jax.numpy as jnp
from jax.experimental import pallas as pl

_CHUNK = 256
_NACC = 2


def _count_lt16_2(va, ca, vb, cb):
    rows, n = va.shape
    one = jnp.int16(1)
    zero = jnp.int16(0)
    acca = [jnp.zeros((rows, _CHUNK), jnp.int16) for _ in range(_NACC)]
    accb = [jnp.zeros((rows, _CHUNK), jnp.int16) for _ in range(_NACC)]
    for i, c in enumerate(range(0, n, _CHUNK)):
        sl = slice(c, c + _CHUNK)
        acca[i % _NACC] = acca[i % _NACC] + jnp.where(va[:, sl] < ca, one, zero)
        accb[i % _NACC] = accb[i % _NACC] + jnp.where(vb[:, sl] < cb, one, zero)
    while len(acca) > 1:
        acca = [x + y for x, y in zip(acca[::2], acca[1::2])]
        accb = [x + y for x, y in zip(accb[::2], accb[1::2])]
    cnta = jnp.sum(acca[0].astype(jnp.int32), axis=1, keepdims=True)
    cntb = jnp.sum(accb[0].astype(jnp.int32), axis=1, keepdims=True)
    return cnta, cntb


def _rowminf(vals):
    x = vals
    while x.shape[1] > _CHUNK:
        half = x.shape[1] // 2
        x = jnp.minimum(x[:, :half], x[:, half:])
    return jnp.min(x, axis=1, keepdims=True)


def _count_lef(vals, v):
    rows, n = vals.shape
    one = jnp.float32(1.0)
    zero = jnp.float32(0.0)
    accs = [jnp.zeros((rows, _CHUNK), jnp.float32) for _ in range(_NACC)]
    for i, c in enumerate(range(0, n, _CHUNK)):
        accs[i % _NACC] = accs[i % _NACC] + jnp.where(
            vals[:, c:c + _CHUNK] <= v, one, zero)
    while len(accs) > 1:
        accs = [x + y for x, y in zip(accs[::2], accs[1::2])]
    return jnp.sum(accs[0], axis=1, keepdims=True)


def _ktakes_kernel(k, g_ref, out_ref):
    g = g_ref[...]
    b = jax.lax.bitcast_convert_type(g, jnp.uint32)
    rows = g.shape[0]
    half = rows // 2
    xk = jnp.uint32(0x8000)

    hw = b >> jnp.uint32(16)
    hkey = jnp.where(b >= jnp.uint32(0x80000000),
                     jnp.uint32(0xFFFF) - hw, hw | jnp.uint32(0x8000))
    h = (hkey ^ xk).astype(jnp.int16)
    ha, hb = h[:half], h[half:]

    thra = jnp.zeros((half, 1), jnp.uint32)
    thrb = jnp.zeros((rows - half, 1), jnp.uint32)
    for bit in range(15, -1, -1):
        canda = thra | jnp.uint32(1 << bit)
        candb = thrb | jnp.uint32(1 << bit)
        cnta, cntb = _count_lt16_2(ha, (canda ^ xk).astype(jnp.int16),
                                   hb, (candb ^ xk).astype(jnp.int16))
        thra = jnp.where(cnta < k, canda, thra)
        thrb = jnp.where(cntb < k, candb, thrb)
    thr = jnp.concatenate([thra, thrb], axis=0)

    thr_s = (thr ^ xk).astype(jnp.int16)
    basea, baseb = _count_lt16_2(ha, thr_s[:half], hb, thr_s[half:])
    k2 = (k - jnp.concatenate([basea, baseb], axis=0)).astype(jnp.float32)
    af = jnp.where(h == thr_s, g, jnp.float32(jnp.inf))

    v0 = _rowminf(af)
    c0 = _count_lef(af, v0)

    def cond(state):
        v, ccum, tf = state
        return jnp.any(ccum < k2)

    def body(state):
        v, ccum, tf = state
        nxt = _rowminf(jnp.where(af > v, af, jnp.float32(jnp.inf)))
        newc = _count_lef(af, nxt)
        upd = ccum < k2
        tf = jnp.where(upd, nxt, tf)
        ccum = jnp.where(upd, newc, ccum)
        return nxt, ccum, tf

    _, _, tf = jax.lax.while_loop(cond, body, (v0, c0, v0))

    out_ref[...] = jnp.where(g <= tf, jnp.float32(0.0), g)


def kernel(g):
    B, N = g.shape
    k = int(N * 0.5)
    return pl.pallas_call(
        functools.partial(_ktakes_kernel, k),
        out_shape=jax.ShapeDtypeStruct((B, N), g.dtype),
    )(g)


# speedup vs baseline: 53.1067x; 1.1655x over previous
import functools

import jax
import jax.numpy as jnp
from jax.experimental import pallas as pl

_CHUNK = 256
_NACC = 2


def _count_lt16_2(va, ca, vb, cb):
    rows, n = va.shape
    one = jnp.int16(1)
    zero = jnp.int16(0)
    acca = [jnp.zeros((rows, _CHUNK), jnp.int16) for _ in range(_NACC)]
    accb = [jnp.zeros((rows, _CHUNK), jnp.int16) for _ in range(_NACC)]
    for i, c in enumerate(range(0, n, _CHUNK)):
        sl = slice(c, c + _CHUNK)
        acca[i % _NACC] = acca[i % _NACC] + jnp.where(va[:, sl] < ca, one, zero)
        accb[i % _NACC] = accb[i % _NACC] + jnp.where(vb[:, sl] < cb, one, zero)
    while len(acca) > 1:
        acca = [x + y for x, y in zip(acca[::2], acca[1::2])]
        accb = [x + y for x, y in zip(accb[::2], accb[1::2])]
    cnta = jnp.sum(acca[0].astype(jnp.float32), axis=1, keepdims=True)
    cntb = jnp.sum(accb[0].astype(jnp.float32), axis=1, keepdims=True)
    return cnta, cntb


def _rowminf(vals):
    x = vals
    while x.shape[1] > _CHUNK:
        half = x.shape[1] // 2
        x = jnp.minimum(x[:, :half], x[:, half:])
    return jnp.min(x, axis=1, keepdims=True)


def _count_lef(vals, v):
    rows, n = vals.shape
    one = jnp.float32(1.0)
    zero = jnp.float32(0.0)
    accs = [jnp.zeros((rows, _CHUNK), jnp.float32) for _ in range(_NACC)]
    for i, c in enumerate(range(0, n, _CHUNK)):
        accs[i % _NACC] = accs[i % _NACC] + jnp.where(
            vals[:, c:c + _CHUNK] <= v, one, zero)
    while len(accs) > 1:
        accs = [x + y for x, y in zip(accs[::2], accs[1::2])]
    return jnp.sum(accs[0], axis=1, keepdims=True)


def _ktakes_kernel(k, g_ref, out_ref):
    g = g_ref[...]
    b = jax.lax.bitcast_convert_type(g, jnp.uint32)
    rows = g.shape[0]
    half = rows // 2
    xk = jnp.uint32(0x8000)

    hw = b >> jnp.uint32(16)
    hkey = jnp.where(b >= jnp.uint32(0x80000000),
                     jnp.uint32(0xFFFF) - hw, hw | jnp.uint32(0x8000))
    h = (hkey ^ xk).astype(jnp.int16)
    ha, hb = h[:half], h[half:]

    kf = jnp.float32(k)
    thra = jnp.zeros((half, 1), jnp.uint32)
    thrb = jnp.zeros((rows - half, 1), jnp.uint32)
    basea = jnp.zeros((half, 1), jnp.float32)
    baseb = jnp.zeros((rows - half, 1), jnp.float32)
    for bit in range(15, -1, -1):
        canda = thra | jnp.uint32(1 << bit)
        candb = thrb | jnp.uint32(1 << bit)
        cnta, cntb = _count_lt16_2(ha, (canda ^ xk).astype(jnp.int16),
                                   hb, (candb ^ xk).astype(jnp.int16))
        taka = cnta < kf
        takb = cntb < kf
        thra = jnp.where(taka, canda, thra)
        thrb = jnp.where(takb, candb, thrb)
        basea = jnp.where(taka, cnta, basea)
        baseb = jnp.where(takb, cntb, baseb)
    thr = jnp.concatenate([thra, thrb], axis=0)

    thr_s = (thr ^ xk).astype(jnp.int16)
    k2 = kf - jnp.concatenate([basea, baseb], axis=0)
    af = jnp.where(h == thr_s, g, jnp.float32(jnp.inf))

    v0 = _rowminf(af)
    c0 = _count_lef(af, v0)

    def cond(state):
        v, ccum, tf = state
        return jnp.any(ccum < k2)

    def body(state):
        v, ccum, tf = state
        nxt = _rowminf(jnp.where(af > v, af, jnp.float32(jnp.inf)))
        newc = _count_lef(af, nxt)
        upd = ccum < k2
        tf = jnp.where(upd, nxt, tf)
        ccum = jnp.where(upd, newc, ccum)
        return nxt, ccum, tf

    _, _, tf = jax.lax.while_loop(cond, body, (v0, c0, v0))

    out_ref[...] = jnp.where(g <= tf, jnp.float32(0.0), g)


def kernel(g):
    B, N = g.shape
    k = int(N * 0.5)
    return pl.pallas_call(
        functools.partial(_ktakes_kernel, k),
        out_shape=jax.ShapeDtypeStruct((B, N), g.dtype),
    )(g)


# speedup vs baseline: 56.0653x; 1.0557x over previous
import functools

import jax
import jax.numpy as jnp
from jax.experimental import pallas as pl

_CHUNK = 256
_NACC = 2


def _count_lt16_2(va, ca, vb, cb):
    rows, n = va.shape
    one = jnp.int16(1)
    zero = jnp.int16(0)
    acca = [jnp.zeros((rows, _CHUNK), jnp.int16) for _ in range(_NACC)]
    accb = [jnp.zeros((rows, _CHUNK), jnp.int16) for _ in range(_NACC)]
    for i, c in enumerate(range(0, n, _CHUNK)):
        sl = slice(c, c + _CHUNK)
        acca[i % _NACC] = acca[i % _NACC] + jnp.where(va[:, sl] < ca, one, zero)
        accb[i % _NACC] = accb[i % _NACC] + jnp.where(vb[:, sl] < cb, one, zero)
    while len(acca) > 1:
        acca = [x + y for x, y in zip(acca[::2], acca[1::2])]
        accb = [x + y for x, y in zip(accb[::2], accb[1::2])]
    cnta = jnp.sum(acca[0].astype(jnp.float32), axis=1, keepdims=True)
    cntb = jnp.sum(accb[0].astype(jnp.float32), axis=1, keepdims=True)
    return cnta, cntb


def _rowminf(vals):
    x = vals
    while x.shape[1] > _CHUNK:
        half = x.shape[1] // 2
        x = jnp.minimum(x[:, :half], x[:, half:])
    return jnp.min(x, axis=1, keepdims=True)


def _count_lef(vals, v):
    rows, n = vals.shape
    one = jnp.float32(1.0)
    zero = jnp.float32(0.0)
    accs = [jnp.zeros((rows, _CHUNK), jnp.float32) for _ in range(_NACC)]
    for i, c in enumerate(range(0, n, _CHUNK)):
        accs[i % _NACC] = accs[i % _NACC] + jnp.where(
            vals[:, c:c + _CHUNK] <= v, one, zero)
    while len(accs) > 1:
        accs = [x + y for x, y in zip(accs[::2], accs[1::2])]
    return jnp.sum(accs[0], axis=1, keepdims=True)


def _ktakes_kernel(k, g_ref, out_ref):
    g = g_ref[...]
    b = jax.lax.bitcast_convert_type(g, jnp.uint32)
    rows = g.shape[0]
    half = rows // 2
    xk = jnp.uint32(0x8000)

    hw = b >> jnp.uint32(16)
    hkey = jnp.where(b >= jnp.uint32(0x80000000),
                     jnp.uint32(0xFFFF) - hw, hw | jnp.uint32(0x8000))
    h = (hkey ^ xk).astype(jnp.int16)
    ha, hb = h[:half], h[half:]

    kf = jnp.float32(k)
    thra = jnp.zeros((half, 1), jnp.uint32)
    thrb = jnp.zeros((rows - half, 1), jnp.uint32)
    basea = jnp.zeros((half, 1), jnp.float32)
    baseb = jnp.zeros((rows - half, 1), jnp.float32)
    for bit in range(15, -1, -1):
        canda = thra | jnp.uint32(1 << bit)
        candb = thrb | jnp.uint32(1 << bit)
        cnta, cntb = _count_lt16_2(ha, (canda ^ xk).astype(jnp.int16),
                                   hb, (candb ^ xk).astype(jnp.int16))
        taka = cnta < kf
        takb = cntb < kf
        thra = jnp.where(taka, canda, thra)
        thrb = jnp.where(takb, candb, thrb)
        basea = jnp.where(taka, cnta, basea)
        baseb = jnp.where(takb, cntb, baseb)
    thr = jnp.concatenate([thra, thrb], axis=0)

    k2 = kf - jnp.concatenate([basea, baseb], axis=0)
    af = jnp.where(hkey == thr, g, jnp.float32(jnp.inf))

    v0 = _rowminf(af)
    c0 = _count_lef(af, v0)

    def cond(state):
        v, ccum, tf = state
        return jnp.any(ccum < k2)

    def body(state):
        v, ccum, tf = state
        nxt = _rowminf(jnp.where(af > v, af, jnp.float32(jnp.inf)))
        newc = _count_lef(af, nxt)
        upd = ccum < k2
        tf = jnp.where(upd, nxt, tf)
        ccum = jnp.where(upd, newc, ccum)
        return nxt, ccum, tf

    _, _, tf = jax.lax.while_loop(cond, body, (v0, c0, v0))

    out_ref[...] = jnp.where(g <= tf, jnp.float32(0.0), g)


def kernel(g):
    B, N = g.shape
    k = int(N * 0.5)
    return pl.pallas_call(
        functools.partial(_ktakes_kernel, k),
        out_shape=jax.ShapeDtypeStruct((B, N), g.dtype),
    )(g)
